# baseline (speedup 1.0000x reference)
import jax
import jax.numpy as jnp
from jax import lax
from jax.experimental import pallas as pl
from jax.experimental.pallas import tpu as pltpu
from jax.experimental.pallas import tpu_sc as plsc

NC = 2
NS = 16
NW = NC * NS
LANES = 16
EMBED = 64
BLK = 128
NBUF = 4
SCALE = 8.0


NPB = 7840
RBUF = 4
PBUF = 5
VPAD = 5


def _make_panelize_kernel(v):
    mesh = plsc.VectorSubcoreMesh(core_axis_name="c", subcore_axis_name="s")
    n_blocks = (v + BLK - 1) // BLK
    n_full = n_blocks // NW
    last_blk = n_blocks - 1

    def body(tab_hbm, tail_hbm, out_hbm, *sems):
        wid = lax.axis_index("s") * NC + lax.axis_index("c")

        def copy(blk, slot):
            pltpu.async_copy(tab_hbm.at[:, pl.ds(blk * BLK, BLK)],
                             out_hbm.at[blk], sems[slot])

        def copy_wait(slot):
            pltpu.make_async_copy(tab_hbm.at[:, pl.ds(0, BLK)],
                                  out_hbm.at[0], sems[slot]).wait()

        for slot in range(RBUF):
            copy(wid + slot * NW, slot)

        n_groups = n_full // RBUF

        def group_body(g, carry):
            for slot in range(RBUF):
                j = g * RBUF + slot
                copy_wait(slot)

                @pl.when(g < n_groups - 1)
                def _():
                    copy(wid + (j + RBUF) * NW, slot)
            return carry

        lax.fori_loop(0, n_groups, group_body, 0)

        @pl.when(wid < n_blocks - n_full * NW)
        def _():
            blk = n_full * NW + wid

            @pl.when(blk != last_blk)
            def _():
                pltpu.sync_copy(tab_hbm.at[:, pl.ds(blk * BLK, BLK)],
                                out_hbm.at[blk])

            @pl.when(blk == last_blk)
            def _():
                pltpu.sync_copy(tail_hbm, out_hbm.at[last_blk])

    return pl.kernel(
        body,
        out_type=jax.ShapeDtypeStruct((NPB, EMBED, BLK), jnp.float32),
        mesh=mesh,
        scratch_types=[pltpu.SemaphoreType.DMA] * RBUF,
        compiler_params=pltpu.CompilerParams(use_tc_tiling_on_sc=True,
                                             needs_layout_passes=False),
    )


def _make_pack_kernel():
    mesh = plsc.VectorSubcoreMesh(core_axis_name="c", subcore_axis_name="s")
    n_per_w = NPB // NW
    stage_w = 2 * EMBED + VPAD

    def body(pan_hbm, out_hbm, *bufs):
        in_v = bufs[:PBUF]
        st_v = bufs[PBUF:2 * PBUF]
        gsems = bufs[2 * PBUF:3 * PBUF]
        ssems = bufs[3 * PBUF:4 * PBUF]
        wid = lax.axis_index("s") * NC + lax.axis_index("c")

        def gather(blk, slot):
            pltpu.async_copy(pan_hbm.at[blk], in_v[slot], gsems[slot])

        def gather_wait(slot):
            pltpu.make_async_copy(pan_hbm.at[0], in_v[slot],
                                  gsems[slot]).wait()

        def scatter(blk, slot):
            pltpu.async_copy(st_v[slot].at[:, pl.ds(0, 2 * EMBED)],
                             out_hbm.at[pl.ds(blk * (BLK // 2), BLK // 2)],
                             ssems[slot])

        def scatter_wait(slot):
            pltpu.make_async_copy(st_v[slot].at[:, pl.ds(0, 2 * EMBED)],
                                  out_hbm.at[pl.ds(0, BLK // 2)],
                                  ssems[slot]).wait()

        base = lax.iota(jnp.int32, 16)
        vloc = [base + (g * LANES) for g in range(BLK // LANES)]
        row_g = [vl >> 1 for vl in vloc]
        colb_g = [(vl & 1) * EMBED for vl in vloc]

        def transpose(slot):
            @plsc.parallel_loop(0, EMBED, 1, unroll=2)
            def _(d):
                for g in range(BLK // LANES):
                    vv = in_v[slot][d, pl.ds(g * LANES, LANES)]
                    plsc.store_scatter(st_v[slot], [row_g[g], colb_g[g] + d],
                                       vv)

        for slot in range(PBUF):
            gather(wid + slot * NW, slot)

        n_groups = n_per_w // PBUF

        def group_body(g, carry):
            for slot in range(PBUF):
                j = g * PBUF + slot
                gather_wait(slot)

                @pl.when(g >= 1)
                def _():
                    scatter_wait(slot)

                transpose(slot)

                @pl.when(g < n_groups - 1)
                def _():
                    gather(wid + (j + PBUF) * NW, slot)

                scatter(wid + j * NW, slot)
            return carry

        lax.fori_loop(0, n_groups, group_body, 0)
        for slot in range(PBUF):
            scatter_wait(slot)

    return pl.kernel(
        body,
        out_type=jax.ShapeDtypeStruct((NPB * EMBED, 2 * EMBED), jnp.float32),
        mesh=mesh,
        scratch_types=(
            [pltpu.VMEM((EMBED, BLK), jnp.float32)] * PBUF
            + [pltpu.VMEM((BLK // 2, stage_w), jnp.float32)] * PBUF
            + [pltpu.SemaphoreType.DMA] * (2 * PBUF)
        ),
        compiler_params=pltpu.CompilerParams(use_tc_tiling_on_sc=False,
                                             needs_layout_passes=False),
    )


def _make_sc_kernel(b, t):
    mesh = plsc.VectorSubcoreMesh(core_axis_name="c", subcore_axis_name="s")
    n_blk = b // BLK
    assert n_blk == NW
    n_tr = t // 8
    n_units = t

    def body(idx_hbm, table_hbm, out_hbm, idx_v, *bufs):
        in_v = bufs[:NBUF]
        out_v = bufs[NBUF:2 * NBUF]
        gsems = bufs[2 * NBUF:3 * NBUF]
        ssems = bufs[3 * NBUF:4 * NBUF]
        wid = lax.axis_index("s") * NC + lax.axis_index("c")
        pltpu.sync_copy(idx_hbm.at[:, wid], idx_v)

        def gather(u, slot):
            i = u // 8
            s = u - i * 8
            pltpu.async_copy(table_hbm.at[idx_v.at[i, s]], in_v[slot],
                             gsems[slot])

        def gather_wait(slot):
            pltpu.make_async_copy(
                table_hbm.at[idx_v.at[0, 0]], in_v[slot], gsems[slot]).wait()

        def scatter(u, slot):
            for dk in range(EMBED // 8):
                pltpu.async_copy(out_v[slot].at[pl.ds(dk * 8, 8), pl.ds(0, BLK)],
                                 out_hbm.at[u, dk, wid], ssems[slot])

        def scatter_wait(u, slot):
            for dk in range(EMBED // 8):
                pltpu.make_async_copy(
                    out_v[slot].at[pl.ds(dk * 8, 8), pl.ds(0, BLK)],
                    out_hbm.at[u, dk, wid], ssems[slot]).wait()

        base = lax.iota(jnp.int32, 16)
        d_ids = [base + (k * LANES) for k in range(EMBED // LANES)]

        for slot in range(NBUF):
            gather(slot, slot)

        n_groups = n_units // NBUF

        def group_body(g, carry):
            for slot in range(NBUF):
                u = g * NBUF + slot
                gather_wait(slot)

                @pl.when(g >= 1)
                def _():
                    scatter_wait(u - NBUF, slot)

                @plsc.parallel_loop(0, BLK, 1, unroll=2)
                def _(l):
                    tok = jnp.broadcast_to(l, (16,)).astype(jnp.int32)
                    for k in range(EMBED // LANES):
                        v = in_v[slot][l, pl.ds(k * LANES, LANES)]
                        plsc.store_scatter(out_v[slot], [d_ids[k], tok],
                                           v * SCALE)

                @pl.when(g < n_groups - 1)
                def _():
                    gather(u + NBUF, slot)

                scatter(u, slot)
            return carry

        lax.fori_loop(0, n_groups, group_body, 0)

        for slot in range(NBUF):
            scatter_wait(n_units - NBUF + slot, slot)

    return pl.kernel(
        body,
        out_type=jax.ShapeDtypeStruct((t, EMBED // 8, NW, 8, BLK), jnp.float32),
        mesh=mesh,
        scratch_types=(
            [pltpu.VMEM((n_tr, 8, BLK), jnp.int32)]
            + [pltpu.VMEM((BLK, EMBED), jnp.float32)] * NBUF
            + [pltpu.VMEM((EMBED, BLK + 5), jnp.float32)] * NBUF
            + [pltpu.SemaphoreType.DMA] * (2 * NBUF)
        ),
        compiler_params=pltpu.CompilerParams(use_tc_tiling_on_sc=False,
                                             needs_layout_passes=False),
    )


def kernel(input, table):
    b, t = input.shape
    v = table.shape[0]
    n_full_cols = (v // BLK) * BLK
    tail = jnp.pad(table[n_full_cols:].T,
                   ((0, 0), (0, BLK - (v - n_full_cols))))
    panels = _make_panelize_kernel(v)(table.T, tail)
    table_rm = _make_pack_kernel()(panels).reshape(NPB * BLK, EMBED)
    idx_view = (input.astype(jnp.int32).T
                .reshape(t // 8, 8, b // BLK, BLK)
                .transpose(0, 2, 1, 3))
    out5 = _make_sc_kernel(b, t)(idx_view, table_rm)
    return out5.transpose(2, 4, 0, 1, 3).reshape(b, t, EMBED)


# speedup vs baseline: 10.6586x; 10.6586x over previous
import jax
import jax.numpy as jnp
from jax import lax
from jax.experimental import pallas as pl
from jax.experimental.pallas import tpu as pltpu
from jax.experimental.pallas import tpu_sc as plsc

NC = 2
NS = 16
NW = NC * NS
LANES = 16
EMBED = 64
BLK = 128
NBUF = 4
SCALE = 8.0


def _make_sc_kernel(b, t):
    mesh = plsc.VectorSubcoreMesh(core_axis_name="c", subcore_axis_name="s")
    n_blk = b // BLK
    assert n_blk == NW
    n_tr = t // 8
    n_units = t

    def body(idx_hbm, table_hbm, out_hbm, idx_v, *bufs):
        in_v = bufs[:NBUF]
        out_v = bufs[NBUF:2 * NBUF]
        gsems = bufs[2 * NBUF:3 * NBUF]
        ssems = bufs[3 * NBUF:4 * NBUF]
        wid = lax.axis_index("s") * NC + lax.axis_index("c")
        pltpu.sync_copy(idx_hbm.at[:, wid], idx_v)

        def gather(u, slot):
            i = u // 8
            s = u - i * 8
            pltpu.async_copy(table_hbm.at[idx_v.at[i, s]], in_v[slot],
                             gsems[slot])

        def gather_wait(slot):
            pltpu.make_async_copy(
                table_hbm.at[idx_v.at[0, 0]], in_v[slot], gsems[slot]).wait()

        def scatter(u, slot):
            for dk in range(EMBED // 8):
                pltpu.async_copy(out_v[slot].at[pl.ds(dk * 8, 8), pl.ds(0, BLK)],
                                 out_hbm.at[u, dk, wid], ssems[slot])

        def scatter_wait(u, slot):
            for dk in range(EMBED // 8):
                pltpu.make_async_copy(
                    out_v[slot].at[pl.ds(dk * 8, 8), pl.ds(0, BLK)],
                    out_hbm.at[u, dk, wid], ssems[slot]).wait()

        base = lax.iota(jnp.int32, 16)
        d_ids = [base + (k * LANES) for k in range(EMBED // LANES)]

        for slot in range(NBUF):
            gather(slot, slot)

        n_groups = n_units // NBUF

        def group_body(g, carry):
            for slot in range(NBUF):
                u = g * NBUF + slot
                gather_wait(slot)

                @pl.when(g >= 1)
                def _():
                    scatter_wait(u - NBUF, slot)

                @plsc.parallel_loop(0, BLK, 1, unroll=2)
                def _(l):
                    tok = jnp.broadcast_to(l, (16,)).astype(jnp.int32)
                    for k in range(EMBED // LANES):
                        v = in_v[slot][l, pl.ds(k * LANES, LANES)]
                        plsc.store_scatter(out_v[slot], [d_ids[k], tok],
                                           v * SCALE)

                @pl.when(g < n_groups - 1)
                def _():
                    gather(u + NBUF, slot)

                scatter(u, slot)
            return carry

        lax.fori_loop(0, n_groups, group_body, 0)

        for slot in range(NBUF):
            scatter_wait(n_units - NBUF + slot, slot)

    return pl.kernel(
        body,
        out_type=jax.ShapeDtypeStruct((t, EMBED // 8, NW, 8, BLK), jnp.float32),
        mesh=mesh,
        scratch_types=(
            [pltpu.VMEM((n_tr, 8, BLK), jnp.int32)]
            + [pltpu.VMEM((BLK, EMBED), jnp.float32)] * NBUF
            + [pltpu.VMEM((EMBED, BLK + 5), jnp.float32)] * NBUF
            + [pltpu.SemaphoreType.DMA] * (2 * NBUF)
        ),
        compiler_params=pltpu.CompilerParams(use_tc_tiling_on_sc=False,
                                             needs_layout_passes=False),
    )


def kernel(input, table):
    b, t = input.shape
    idx_view = (input.astype(jnp.int32).T
                .reshape(t // 8, 8, b // BLK, BLK)
                .transpose(0, 2, 1, 3))
    out5 = _make_sc_kernel(b, t)(idx_view, table)
    return out5.transpose(2, 4, 0, 1, 3).reshape(b, t, EMBED)
